# baseline (speedup 1.0000x reference)
import functools

import jax
import jax.numpy as jnp
from jax import lax
from jax.experimental import pallas as pl
from jax.experimental.pallas import tpu as pltpu
from jax.experimental.pallas import tpu_sc as plsc

_N = 320000
_C = 128
_B = 8
_NC = 2
_NS = 16
_NW = _NC * _NS
_R = 200
_SPLIT = 153600
_CHUNK = _SPLIT // _NW
_NBLK = _CHUNK // _R
_NVEC = _CHUNK // 16
_CG = _C // 16
_TBLK = 6400
_TGRID = (_N - _SPLIT) // _TBLK


def _sc_partial(feats, bidx):
    mesh = plsc.VectorSubcoreMesh(core_axis_name="c", subcore_axis_name="s")

    @functools.partial(
        pl.kernel,
        mesh=mesh,
        out_type=[
            jax.ShapeDtypeStruct((_NW, _B, _C), jnp.float32),
            jax.ShapeDtypeStruct((_NW, _B, _C), jnp.float32),
        ],
        scratch_types=[
            pltpu.VMEM((_R, _C), jnp.float32),
            pltpu.VMEM((_R, _C), jnp.float32),
            pltpu.VMEM((_CHUNK,), jnp.int32),
            pltpu.VMEM((_B, _C), jnp.float32),
            pltpu.VMEM((_B, _C), jnp.float32),
            pltpu.SemaphoreType.DMA,
            pltpu.SemaphoreType.DMA,
            pltpu.SemaphoreType.DMA,
        ],
    )
    def k(feats_hbm, bidx_hbm, psum_hbm, pcnt_hbm,
          buf0, buf1, bidx_v, acc, cntb, sem0, sem1, semi):
        wid = lax.axis_index("c") * _NS + lax.axis_index("s")
        base = wid * _CHUNK

        cpi = pltpu.async_copy(bidx_hbm.at[pl.ds(base, _CHUNK)], bidx_v, semi)
        pltpu.async_copy(feats_hbm.at[pl.ds(base, _R), :], buf0, sem0)
        pltpu.async_copy(feats_hbm.at[pl.ds(base + _R, _R), :], buf1, sem1)
        cpi.wait()

        def lower_bound(target):
            def body(_, lohi):
                lo, hi = lohi
                mid = jnp.minimum((lo + hi) // 2, _NVEC - 1)
                key = bidx_v[pl.ds(mid * 16, 16)][0]
                active = lo < hi
                right = jnp.logical_and(active, key < target)
                left = jnp.logical_and(active, key >= target)
                return (jnp.where(right, mid + 1, lo),
                        jnp.where(left, mid, hi))

            g, _ = lax.fori_loop(
                0, max(_NVEC.bit_length(), 1), body,
                (jnp.int32(0), jnp.int32(_NVEC)))
            gx = jnp.maximum(g, 1) - 1
            v = bidx_v[pl.ds(gx * 16, 16)]
            cnt = jnp.int32(0)
            for lane in range(16):
                cnt = cnt + jnp.where(v[lane] < target, 1, 0).astype(jnp.int32)
            return gx * 16 + cnt

        starts = (
            [jnp.int32(0)]
            + [lower_bound(jnp.int32(s)) for s in range(1, _B)]
            + [jnp.int32(_CHUNK)]
        )

        zv = jnp.zeros((16,), jnp.float32)
        for s in range(_B):
            for j in range(_CG):
                acc[s, pl.ds(j * 16, 16)] = zv

        def process(buf, blk_row):
            for s in range(_B):
                lo = jnp.clip(starts[s] - blk_row, 0, _R)
                hi = jnp.clip(starts[s + 1] - blk_row, 0, _R)

                def rbody(i, a):
                    return tuple(
                        a[j] + buf[i, pl.ds(j * 16, 16)] for j in range(_CG)
                    )

                a0 = tuple(acc[s, pl.ds(j * 16, 16)] for j in range(_CG))
                a = lax.fori_loop(lo, hi, rbody, a0)
                for j in range(_CG):
                    acc[s, pl.ds(j * 16, 16)] = a[j]

        def gbody(g, carry):
            b0 = 2 * g
            r0 = base + b0 * _R
            pltpu.make_async_copy(
                feats_hbm.at[pl.ds(r0, _R), :], buf0, sem0).wait()
            process(buf0, b0 * _R)

            @pl.when(b0 + 2 < _NBLK)
            def _():
                pltpu.async_copy(
                    feats_hbm.at[pl.ds(r0 + 2 * _R, _R), :], buf0, sem0)

            pltpu.make_async_copy(
                feats_hbm.at[pl.ds(r0 + _R, _R), :], buf1, sem1).wait()
            process(buf1, (b0 + 1) * _R)

            @pl.when(b0 + 3 < _NBLK)
            def _():
                pltpu.async_copy(
                    feats_hbm.at[pl.ds(r0 + 3 * _R, _R), :], buf1, sem1)

            return carry

        lax.fori_loop(0, _NBLK // 2, gbody, jnp.int32(0))

        for s in range(_B):
            c = (starts[s + 1] - starts[s]).astype(jnp.float32)
            vec = jnp.full((16,), c, jnp.float32)
            for j in range(_CG):
                cntb[s, pl.ds(j * 16, 16)] = vec

        pltpu.sync_copy(acc, psum_hbm.at[wid])
        pltpu.sync_copy(cntb, pcnt_hbm.at[wid])

    return k(feats, bidx)


def _tc_body(bidx_ref, feats_ref, psum_ref, pcnt_ref, q_ref):
    i = pl.program_id(0)

    @pl.when(i == 0)
    def _():
        psum_ref[...] = jnp.zeros((_B, _C), jnp.float32)
        pcnt_ref[...] = jnp.zeros((_B, _C), jnp.float32)

    bvec = bidx_ref[0, 0, :]
    first = bvec[0]
    last = bvec[_TBLK - 1]
    feats = feats_ref[...]
    total = jnp.sum(feats, axis=0)

    rowid = lax.broadcasted_iota(jnp.int32, (_TBLK, _C), 0)
    q_ref[0, :] = jnp.zeros((_C,), jnp.float32)
    q_ref[_B, :] = total
    cnts = [jnp.float32(0.0)]
    for s in range(1, _B):
        c = jnp.sum((bvec < s).astype(jnp.float32))
        cnts.append(c)

        @pl.when(first >= s)
        def _():
            q_ref[s, :] = jnp.zeros((_C,), jnp.float32)

        @pl.when(last < s)
        def _():
            q_ref[s, :] = total

        @pl.when(jnp.logical_and(first < s, last >= s))
        def _():
            m = rowid < c.astype(jnp.int32)
            q_ref[s, :] = jnp.sum(jnp.where(m, feats, 0.0), axis=0)

    cnts.append(jnp.float32(_TBLK))
    cnt = jnp.stack([cnts[s + 1] - cnts[s] for s in range(_B)])

    psum_ref[...] += q_ref[1:, :] - q_ref[:_B, :]
    pcnt_ref[...] += jnp.broadcast_to(cnt[:, None], (_B, _C))


def _tc_partial(feats, bidx3):
    return pl.pallas_call(
        _tc_body,
        grid=(_TGRID,),
        in_specs=[
            pl.BlockSpec(
                (1, 1, _TBLK), lambda i: (_SPLIT // _TBLK + i, 0, 0)),
            pl.BlockSpec((_TBLK, _C), lambda i: (_SPLIT // _TBLK + i, 0)),
        ],
        out_specs=[
            pl.BlockSpec((_B, _C), lambda i: (0, 0)),
            pl.BlockSpec((_B, _C), lambda i: (0, 0)),
        ],
        out_shape=[
            jax.ShapeDtypeStruct((_B, _C), jnp.float32),
            jax.ShapeDtypeStruct((_B, _C), jnp.float32),
        ],
        scratch_shapes=[pltpu.VMEM((_B + 1, _C), jnp.float32)],
    )(bidx3, feats)


def _combine_body(ps_ref, cs_ref, tp_ref, tcn_ref, o_ref):
    s = jnp.sum(ps_ref[...], axis=0) + tp_ref[...]
    c = jnp.sum(cs_ref[...], axis=0) + tcn_ref[...]
    o_ref[...] = s / jnp.maximum(c, 1.0)


def _combine(psum, pcnt, tpsum, tpcnt):
    return pl.pallas_call(
        _combine_body,
        out_shape=jax.ShapeDtypeStruct((_B, _C), jnp.float32),
    )(psum, pcnt, tpsum, tpcnt)


def kernel(feats, bidx):
    psum, pcnt = _sc_partial(feats, bidx)
    bidx3 = bidx.reshape(_N // _TBLK, 1, _TBLK)
    tpsum, tpcnt = _tc_partial(feats, bidx3)
    return _combine(psum, pcnt, tpsum, tpcnt)


# speedup vs baseline: 1.1418x; 1.1418x over previous
import functools

import jax
import jax.numpy as jnp
from jax import lax
from jax.experimental import pallas as pl
from jax.experimental.pallas import tpu as pltpu
from jax.experimental.pallas import tpu_sc as plsc

_N = 320000
_C = 128
_B = 8
_NC = 2
_NS = 16
_NW = _NC * _NS
_R = 200
_SPLIT = 192000
_CHUNK = _SPLIT // _NW
_NBLK = _CHUNK // _R
_NVEC = _CHUNK // 16
_CG = _C // 16
_TBLK = 6400
_TGRID = (_N - _SPLIT) // _TBLK


def _sc_partial(feats, bidx):
    mesh = plsc.VectorSubcoreMesh(core_axis_name="c", subcore_axis_name="s")

    @functools.partial(
        pl.kernel,
        mesh=mesh,
        out_type=[
            jax.ShapeDtypeStruct((_NW, _B, _C), jnp.float32),
            jax.ShapeDtypeStruct((_NW, _B, _C), jnp.float32),
        ],
        scratch_types=[
            pltpu.VMEM((_R, _C), jnp.float32),
            pltpu.VMEM((_R, _C), jnp.float32),
            pltpu.VMEM((_CHUNK,), jnp.int32),
            pltpu.VMEM((_B, _C), jnp.float32),
            pltpu.VMEM((_B, _C), jnp.float32),
            pltpu.SemaphoreType.DMA,
            pltpu.SemaphoreType.DMA,
            pltpu.SemaphoreType.DMA,
        ],
    )
    def k(feats_hbm, bidx_hbm, psum_hbm, pcnt_hbm,
          buf0, buf1, bidx_v, acc, cntb, sem0, sem1, semi):
        wid = lax.axis_index("c") * _NS + lax.axis_index("s")
        base = wid * _CHUNK

        cpi = pltpu.async_copy(bidx_hbm.at[pl.ds(base, _CHUNK)], bidx_v, semi)
        pltpu.async_copy(feats_hbm.at[pl.ds(base, _R), :], buf0, sem0)
        pltpu.async_copy(feats_hbm.at[pl.ds(base + _R, _R), :], buf1, sem1)
        cpi.wait()

        def lower_bound(target):
            def body(_, lohi):
                lo, hi = lohi
                mid = jnp.minimum((lo + hi) // 2, _NVEC - 1)
                key = bidx_v[pl.ds(mid * 16, 16)][0]
                active = lo < hi
                right = jnp.logical_and(active, key < target)
                left = jnp.logical_and(active, key >= target)
                return (jnp.where(right, mid + 1, lo),
                        jnp.where(left, mid, hi))

            g, _ = lax.fori_loop(
                0, max(_NVEC.bit_length(), 1), body,
                (jnp.int32(0), jnp.int32(_NVEC)))
            gx = jnp.maximum(g, 1) - 1
            v = bidx_v[pl.ds(gx * 16, 16)]
            cnt = jnp.int32(0)
            for lane in range(16):
                cnt = cnt + jnp.where(v[lane] < target, 1, 0).astype(jnp.int32)
            return gx * 16 + cnt

        starts = (
            [jnp.int32(0)]
            + [lower_bound(jnp.int32(s)) for s in range(1, _B)]
            + [jnp.int32(_CHUNK)]
        )

        zv = jnp.zeros((16,), jnp.float32)
        for s in range(_B):
            for j in range(_CG):
                acc[s, pl.ds(j * 16, 16)] = zv

        def process(buf, blk_row):
            for s in range(_B):
                lo = jnp.clip(starts[s] - blk_row, 0, _R)
                hi = jnp.clip(starts[s + 1] - blk_row, 0, _R)

                def rbody(i, a):
                    return tuple(
                        a[j] + buf[i, pl.ds(j * 16, 16)] for j in range(_CG)
                    )

                a0 = tuple(acc[s, pl.ds(j * 16, 16)] for j in range(_CG))
                a = lax.fori_loop(lo, hi, rbody, a0)
                for j in range(_CG):
                    acc[s, pl.ds(j * 16, 16)] = a[j]

        def gbody(g, carry):
            b0 = 2 * g
            r0 = base + b0 * _R
            pltpu.make_async_copy(
                feats_hbm.at[pl.ds(r0, _R), :], buf0, sem0).wait()
            process(buf0, b0 * _R)

            @pl.when(b0 + 2 < _NBLK)
            def _():
                pltpu.async_copy(
                    feats_hbm.at[pl.ds(r0 + 2 * _R, _R), :], buf0, sem0)

            pltpu.make_async_copy(
                feats_hbm.at[pl.ds(r0 + _R, _R), :], buf1, sem1).wait()
            process(buf1, (b0 + 1) * _R)

            @pl.when(b0 + 3 < _NBLK)
            def _():
                pltpu.async_copy(
                    feats_hbm.at[pl.ds(r0 + 3 * _R, _R), :], buf1, sem1)

            return carry

        lax.fori_loop(0, _NBLK // 2, gbody, jnp.int32(0))

        for s in range(_B):
            c = (starts[s + 1] - starts[s]).astype(jnp.float32)
            vec = jnp.full((16,), c, jnp.float32)
            for j in range(_CG):
                cntb[s, pl.ds(j * 16, 16)] = vec

        pltpu.sync_copy(acc, psum_hbm.at[wid])
        pltpu.sync_copy(cntb, pcnt_hbm.at[wid])

    return k(feats, bidx)


def _tc_body(bidx_ref, feats_ref, psum_ref, pcnt_ref, q_ref):
    i = pl.program_id(0)

    @pl.when(i == 0)
    def _():
        psum_ref[...] = jnp.zeros((_B, _C), jnp.float32)
        pcnt_ref[...] = jnp.zeros((_B, _C), jnp.float32)

    bvec = bidx_ref[0, 0, :]
    first = bvec[0]
    last = bvec[_TBLK - 1]
    feats = feats_ref[...]
    total = jnp.sum(feats, axis=0)

    rowid = lax.broadcasted_iota(jnp.int32, (_TBLK, _C), 0)
    q_ref[0, :] = jnp.zeros((_C,), jnp.float32)
    q_ref[_B, :] = total
    cnts = [jnp.float32(0.0)]
    for s in range(1, _B):
        c = jnp.sum((bvec < s).astype(jnp.float32))
        cnts.append(c)

        @pl.when(first >= s)
        def _():
            q_ref[s, :] = jnp.zeros((_C,), jnp.float32)

        @pl.when(last < s)
        def _():
            q_ref[s, :] = total

        @pl.when(jnp.logical_and(first < s, last >= s))
        def _():
            m = rowid < c.astype(jnp.int32)
            q_ref[s, :] = jnp.sum(jnp.where(m, feats, 0.0), axis=0)

    cnts.append(jnp.float32(_TBLK))
    cnt = jnp.stack([cnts[s + 1] - cnts[s] for s in range(_B)])

    psum_ref[...] += q_ref[1:, :] - q_ref[:_B, :]
    pcnt_ref[...] += jnp.broadcast_to(cnt[:, None], (_B, _C))


def _tc_partial(feats, bidx3):
    return pl.pallas_call(
        _tc_body,
        grid=(_TGRID,),
        in_specs=[
            pl.BlockSpec(
                (1, 1, _TBLK), lambda i: (_SPLIT // _TBLK + i, 0, 0)),
            pl.BlockSpec((_TBLK, _C), lambda i: (_SPLIT // _TBLK + i, 0)),
        ],
        out_specs=[
            pl.BlockSpec((_B, _C), lambda i: (0, 0)),
            pl.BlockSpec((_B, _C), lambda i: (0, 0)),
        ],
        out_shape=[
            jax.ShapeDtypeStruct((_B, _C), jnp.float32),
            jax.ShapeDtypeStruct((_B, _C), jnp.float32),
        ],
        scratch_shapes=[pltpu.VMEM((_B + 1, _C), jnp.float32)],
    )(bidx3, feats)


def _combine_body(ps_ref, cs_ref, tp_ref, tcn_ref, o_ref):
    s = jnp.sum(ps_ref[...], axis=0) + tp_ref[...]
    c = jnp.sum(cs_ref[...], axis=0) + tcn_ref[...]
    o_ref[...] = s / jnp.maximum(c, 1.0)


def _combine(psum, pcnt, tpsum, tpcnt):
    return pl.pallas_call(
        _combine_body,
        out_shape=jax.ShapeDtypeStruct((_B, _C), jnp.float32),
    )(psum, pcnt, tpsum, tpcnt)


def kernel(feats, bidx):
    psum, pcnt = _sc_partial(feats, bidx)
    bidx3 = bidx.reshape(_N // _TBLK, 1, _TBLK)
    tpsum, tpcnt = _tc_partial(feats, bidx3)
    return _combine(psum, pcnt, tpsum, tpcnt)
